# baseline (speedup 1.0000x reference)
import functools

import jax
import jax.numpy as jnp
from jax import lax
from jax.experimental import pallas as pl
from jax.experimental.pallas import tpu as pltpu
from jax.experimental.pallas import tpu_sc as plsc

BATCH = 4096
SEQ = 200
HIDDEN = 64

NC = 2
NS = 16
NW = NC * NS

TOTAL = BATCH * SEQ
PER_W = TOTAL // NW
CH = 128
CHUNKS = PER_W // CH
NBUF = 4
GRP = 2 * NBUF
NITER = CHUNKS // GRP


_mesh = plsc.VectorSubcoreMesh(
    core_axis_name="c", subcore_axis_name="s", num_cores=NC, num_subcores=NS
)


@functools.partial(
    pl.kernel,
    out_type=jax.ShapeDtypeStruct((TOTAL, HIDDEN), jnp.float32),
    mesh=_mesh,
    compiler_params=pltpu.CompilerParams(use_tc_tiling_on_sc=False),
    scratch_types=[
        pltpu.VMEM((CHUNKS, CH), jnp.int32),
        pltpu.VMEM((2 * NBUF, CH, HIDDEN), jnp.float32),
        pltpu.SemaphoreType.DMA,
        pltpu.SemaphoreType.DMA,
    ],
)
def _sc_gather(x_hbm, table_hbm, out_hbm, idx_v, bufs, sem_g, sem_s):
    wid = lax.axis_index("s") * NC + lax.axis_index("c")
    base = wid * PER_W

    pltpu.sync_copy(x_hbm.at[wid], idx_v)

    def gather_copy(j, b):
        return pltpu.make_async_copy(
            table_hbm.at[idx_v.at[j]], bufs.at[b], sem_g
        )

    def store_copy(j, b):
        return pltpu.make_async_copy(
            bufs.at[b], out_hbm.at[pl.ds(base + j * CH, CH)], sem_s
        )

    for b in range(NBUF):
        gather_copy(b, b).start()

    def body(h, carry):
        c0 = h * GRP
        for b in range(NBUF):
            gather_copy(c0 + NBUF + b, NBUF + b).start()
        for b in range(NBUF):
            gather_copy(c0 + b, b).wait()
            store_copy(c0 + b, b).start()
        for b in range(NBUF):
            store_copy(c0 + b, b).wait()

        @pl.when(h + 1 < NITER)
        def _():
            for b in range(NBUF):
                gather_copy(c0 + GRP + b, b).start()

        for b in range(NBUF):
            gather_copy(c0 + NBUF + b, NBUF + b).wait()
            store_copy(c0 + NBUF + b, NBUF + b).start()
        for b in range(NBUF):
            store_copy(c0 + NBUF + b, NBUF + b).wait()
        return carry

    lax.fori_loop(0, NITER, body, 0)


def kernel(x, emb_table):
    xf = x.reshape(NW, CHUNKS, CH)
    out = _sc_gather(xf, emb_table)
    return out.reshape(BATCH, SEQ, HIDDEN)
